# baseline (speedup 1.0000x reference)
import functools

import jax
import jax.numpy as jnp
from jax import lax
from jax.experimental import pallas as pl
from jax.experimental.pallas import tpu as pltpu
from jax.experimental.pallas import tpu_sc as plsc

B = 16384
L = 200
DIM = 128
VOCAB = 10

C_SC = 8192
C_TC = B - C_SC
CB = 2048

NC = 2
NS = 16
NW = NC * NS
COLS_W = C_SC // NW
CW = 128
NCH = COLS_W // CW
LANES = 16
RUN = 4


def _lookup16(iv, sv):
    b0 = (iv & 1) != 0
    b1 = (iv & 2) != 0
    b2 = (iv & 4) != 0
    b3 = (iv & 8) != 0
    t01 = jnp.where(b0, sv[1], sv[0])
    t23 = jnp.where(b0, sv[3], sv[2])
    t45 = jnp.where(b0, sv[5], sv[4])
    t67 = jnp.where(b0, sv[7], sv[6])
    t89 = jnp.where(b0, sv[9], sv[8])
    u0 = jnp.where(b1, t23, t01)
    u1 = jnp.where(b1, t67, t45)
    v0 = jnp.where(b2, u1, u0)
    return jnp.where(b3, t89, v0)


def _sc_body(idx_hbm, tabT_hbm, w_hbm, b_hbm, out_hbm,
             idx0_v, idx1_v, out_v, tabT_v, w_v, b_v,
             isem0, isem1, osem):
    pltpu.sync_copy(tabT_hbm, tabT_v)
    pltpu.sync_copy(w_hbm, w_v)
    pltpu.sync_copy(b_hbm, b_v)

    scores = b_v[...]
    for c in range(DIM):
        scores = scores + tabT_v[c, :] * w_v[c, :]
    sv = [jnp.broadcast_to(scores[v], (LANES,)) for v in range(VOCAB)]

    wid = lax.axis_index("s") * NC + lax.axis_index("c")
    col0 = wid * COLS_W

    ibuf = (idx0_v, idx1_v)
    isem = (isem0, isem1)

    def start_in(ci, s):
        c0 = col0 + ci * CW
        return pltpu.async_copy(idx_hbm.at[:, pl.ds(c0, CW)], ibuf[s], isem[s])

    in_cp = {0: start_in(0, 0)}
    out_cp = None
    for ci in range(NCH):
        s = ci & 1
        in_cp[ci].wait()
        if ci + 1 < NCH:
            in_cp[ci + 1] = start_in(ci + 1, 1 - s)
        if out_cp is not None:
            out_cp.wait()
        idx_v = ibuf[s]

        def run_body(r4, c2, idx_v=idx_v):
            for rr in range(RUN):
                r = r4 * RUN + rr
                for off in range(0, CW, LANES):
                    iv = idx_v[r, pl.ds(off, LANES)]
                    out_v[r, pl.ds(off, LANES)] = _lookup16(iv, sv)
            return c2

        lax.fori_loop(0, L // RUN, run_body, 0)
        c0 = col0 + ci * CW
        out_cp = pltpu.async_copy(out_v, out_hbm.at[:, pl.ds(c0, CW)], osem)
    out_cp.wait()


_sc_call = functools.partial(
    pl.kernel,
    out_type=jax.ShapeDtypeStruct((L, B), jnp.float32),
    mesh=plsc.VectorSubcoreMesh(core_axis_name="c", subcore_axis_name="s"),
    scratch_types=[
        pltpu.VMEM((L, CW), jnp.int32),
        pltpu.VMEM((L, CW), jnp.int32),
        pltpu.VMEM((L, CW), jnp.float32),
        pltpu.VMEM((DIM, LANES), jnp.float32),
        pltpu.VMEM((DIM, LANES), jnp.float32),
        pltpu.VMEM((LANES,), jnp.float32),
        pltpu.SemaphoreType.DMA,
        pltpu.SemaphoreType.DMA,
        pltpu.SemaphoreType.DMA,
    ],
)(_sc_body)


def _tc_body(idx_ref, tab_ref, w_ref, b_ref, out_ref):
    scores = jnp.sum(tab_ref[...] * w_ref[...], axis=1) + b_ref[0, 0]
    sv = [scores[v] for v in range(VOCAB)]
    out_ref[...] = _lookup16(idx_ref[...], sv)


_tc_call = pl.pallas_call(
    _tc_body,
    grid=(C_TC // CB,),
    in_specs=[
        pl.BlockSpec((L, CB), lambda i: (0, i + C_SC // CB)),
        pl.BlockSpec((VOCAB, DIM), lambda i: (0, 0)),
        pl.BlockSpec((1, DIM), lambda i: (0, 0)),
        pl.BlockSpec((1, 1), lambda i: (0, 0)),
    ],
    out_specs=pl.BlockSpec((L, CB), lambda i: (0, i)),
    out_shape=jax.ShapeDtypeStruct((L, C_TC), jnp.float32),
)


def kernel(input, table, W, b):
    idxT = input.astype(jnp.int32).T
    tabT = jnp.pad(table.T, ((0, 0), (0, LANES - VOCAB)))
    w16 = jnp.broadcast_to(W.reshape(DIM, 1), (DIM, LANES))
    b16 = jnp.broadcast_to(b, (LANES,))
    sc_out = _sc_call(idxT, tabT, w16, b16)
    tc_out = _tc_call(idxT, table, W, b.reshape(1, 1))
    out = lax.dynamic_update_slice(sc_out, tc_out, (0, C_SC))
    return out.T.reshape(B, L, 1)


# speedup vs baseline: 1.3544x; 1.3544x over previous
import functools

import jax
import jax.numpy as jnp
from jax import lax
from jax.experimental import pallas as pl
from jax.experimental.pallas import tpu as pltpu
from jax.experimental.pallas import tpu_sc as plsc

B = 16384
L = 200
DIM = 128
VOCAB = 10

B_SC = 4096
B_TC = B - B_SC
RB = 512

NC = 2
NS = 16
NW = NC * NS
ROWS_W = B_SC // NW
RCH = 64
NRCH = ROWS_W // RCH
LANES = 16
RUN = 2
OFFS = tuple(range(0, L - LANES + 1, LANES)) + (L - LANES,)


def _lookup16(iv, sv):
    b0 = (iv & 1) != 0
    b1 = (iv & 2) != 0
    b2 = (iv & 4) != 0
    b3 = (iv & 8) != 0
    t01 = jnp.where(b0, sv[1], sv[0])
    t23 = jnp.where(b0, sv[3], sv[2])
    t45 = jnp.where(b0, sv[5], sv[4])
    t67 = jnp.where(b0, sv[7], sv[6])
    t89 = jnp.where(b0, sv[9], sv[8])
    u0 = jnp.where(b1, t23, t01)
    u1 = jnp.where(b1, t67, t45)
    v0 = jnp.where(b2, u1, u0)
    return jnp.where(b3, t89, v0)


def _sc_body(idx_hbm, tabT_hbm, w_hbm, b_hbm, out_hbm,
             idx0_v, idx1_v, out0_v, out1_v, tabT_v, w_v, b_v,
             isem0, isem1, osem0, osem1):
    pltpu.sync_copy(tabT_hbm, tabT_v)
    pltpu.sync_copy(w_hbm, w_v)
    pltpu.sync_copy(b_hbm, b_v)

    scores = b_v[...]
    for c in range(DIM):
        scores = scores + tabT_v[c, :] * w_v[c, :]
    sv = [jnp.broadcast_to(scores[v], (LANES,)) for v in range(VOCAB)]

    wid = lax.axis_index("s") * NC + lax.axis_index("c")
    row0 = wid * ROWS_W

    ibuf = (idx0_v, idx1_v)
    obuf = (out0_v, out1_v)
    isem = (isem0, isem1)
    osem = (osem0, osem1)

    def start_in(ci, s):
        r0 = row0 + ci * RCH
        return pltpu.async_copy(idx_hbm.at[pl.ds(r0, RCH), :], ibuf[s], isem[s])

    def start_out(ci, s):
        r0 = row0 + ci * RCH
        return pltpu.async_copy(obuf[s], out_hbm.at[pl.ds(r0, RCH), :], osem[s])

    in_cp = {0: start_in(0, 0)}
    out_cp = {}
    for ci in range(NRCH):
        s = ci & 1
        in_cp[ci].wait()
        if ci + 1 < NRCH:
            in_cp[ci + 1] = start_in(ci + 1, 1 - s)
        if ci >= 2:
            out_cp[ci - 2].wait()
        idx_v, out_v = ibuf[s], obuf[s]

        def run_body(r2, c2, idx_v=idx_v, out_v=out_v):
            for rr in range(RUN):
                r = r2 * RUN + rr
                for off in OFFS:
                    iv = idx_v[r, pl.ds(off, LANES)]
                    out_v[r, pl.ds(off, LANES)] = _lookup16(iv, sv)
            return c2

        lax.fori_loop(0, RCH // RUN, run_body, 0)
        out_cp[ci] = start_out(ci, s)
    out_cp[NRCH - 2].wait()
    out_cp[NRCH - 1].wait()


_sc_call = functools.partial(
    pl.kernel,
    out_type=jax.ShapeDtypeStruct((B_SC, L), jnp.float32),
    compiler_params=pltpu.CompilerParams(use_tc_tiling_on_sc=True),
    mesh=plsc.VectorSubcoreMesh(core_axis_name="c", subcore_axis_name="s"),
    scratch_types=[
        pltpu.VMEM((RCH, L), jnp.int32),
        pltpu.VMEM((RCH, L), jnp.int32),
        pltpu.VMEM((RCH, L), jnp.float32),
        pltpu.VMEM((RCH, L), jnp.float32),
        pltpu.VMEM((DIM, LANES), jnp.float32),
        pltpu.VMEM((DIM, LANES), jnp.float32),
        pltpu.VMEM((LANES,), jnp.float32),
        pltpu.SemaphoreType.DMA,
        pltpu.SemaphoreType.DMA,
        pltpu.SemaphoreType.DMA,
        pltpu.SemaphoreType.DMA,
    ],
)(_sc_body)


def _tc_body(idx_ref, tab_ref, w_ref, b_ref, out_ref):
    scores = jnp.sum(tab_ref[...] * w_ref[...], axis=1) + b_ref[0, 0]
    sv = [scores[v] for v in range(VOCAB)]
    out_ref[...] = _lookup16(idx_ref[...], sv)


_tc_call = pl.pallas_call(
    _tc_body,
    grid=(B_TC // RB,),
    in_specs=[
        pl.BlockSpec((RB, L), lambda i: (i + B_SC // RB, 0)),
        pl.BlockSpec((VOCAB, DIM), lambda i: (0, 0)),
        pl.BlockSpec((1, DIM), lambda i: (0, 0)),
        pl.BlockSpec((1, 1), lambda i: (0, 0)),
    ],
    out_specs=pl.BlockSpec((RB, L), lambda i: (i + B_SC // RB, 0)),
    out_shape=jax.ShapeDtypeStruct((B, L), jnp.float32),
)


def kernel(input, table, W, b):
    idx = input.astype(jnp.int32)
    tabT = jnp.pad(table.T, ((0, 0), (0, LANES - VOCAB)))
    w16 = jnp.broadcast_to(W.reshape(DIM, 1), (DIM, LANES))
    b16 = jnp.broadcast_to(b, (LANES,))
    sc_out = _sc_call(idx, tabT, w16, b16)
    tc_out = _tc_call(idx, table, W, b.reshape(1, 1))
    out = lax.dynamic_update_slice(tc_out, sc_out, (0, 0))
    return out.reshape(B, L, 1)
